# baseline (speedup 1.0000x reference)
import functools
import jax
import jax.numpy as jnp
from jax import lax
from jax.experimental import pallas as pl
from jax.experimental.pallas import tpu as pltpu
from jax.experimental.pallas import tpu_sc as plsc

_NC = 2
_NS = 16
_NW = _NC * _NS
_D = 64
_SCALE = 8.0
_CHUNK = 40
_NBUF = 4
_TCR = 2048


def _widen_scale(weight_t):
    V = weight_t.shape[1]

    def body(w_ref, o_ref):
        x = w_ref[...]
        eye = (lax.broadcasted_iota(jnp.int32, (_D, _D), 0)
               == lax.broadcasted_iota(jnp.int32, (_D, _D), 1))
        scaled_eye = eye.astype(jnp.float32) * _SCALE
        t = lax.dot_general(x, scaled_eye, (((0,), (0,)), ((), ())),
                            precision=lax.Precision.HIGHEST)
        o_ref[...] = jnp.concatenate([t, t], axis=-1)

    return pl.pallas_call(
        body,
        grid=(pl.cdiv(V, _TCR),),
        in_specs=[pl.BlockSpec((_D, _TCR), lambda i: (0, i))],
        out_specs=pl.BlockSpec((_TCR, 2 * _D), lambda i: (i, 0)),
        out_shape=jax.ShapeDtypeStruct((V, 2 * _D), jnp.float32),
        compiler_params=pltpu.CompilerParams(
            dimension_semantics=("arbitrary",)),
    )(weight_t)


def _make_gather(bsz, seq, V):
    bpw = bsz // _NW
    L = 16
    ng = bpw // L

    mesh = plsc.VectorSubcoreMesh(
        core_axis_name="c", subcore_axis_name="s",
        num_cores=_NC, num_subcores=_NS)

    @functools.partial(
        pl.kernel,
        out_type=jax.ShapeDtypeStruct((seq, _D, bsz), jnp.float32),
        mesh=mesh,
        scratch_types=[
            pltpu.VMEM((seq, bpw), jnp.int32),
            [pltpu.VMEM((bpw, 2 * _D), jnp.float32)] * 2,
            [pltpu.VMEM((_D, bpw), jnp.float32)] * 2,
            [pltpu.SemaphoreType.DMA] * 2,
            [pltpu.SemaphoreType.DMA] * 2,
        ],
        compiler_params=pltpu.CompilerParams(needs_layout_passes=False),
    )
    def emb(idxt_hbm, table_hbm, out_hbm, idx_v, bufs, obufs, gsems, osems):
        wid = lax.axis_index("s") * _NC + lax.axis_index("c")
        b0 = wid * bpw
        pltpu.sync_copy(idxt_hbm.at[:, pl.ds(b0, bpw)], idx_v)

        def fire_gather(s, k):
            pltpu.async_copy(table_hbm.at[idx_v.at[s]], bufs[k], gsems[k])

        def wait_gather(s, k):
            pltpu.make_async_copy(
                table_hbm.at[idx_v.at[s]], bufs[k], gsems[k]).wait()

        def out_slice(s):
            return out_hbm.at[s, :, pl.ds(b0, bpw)]

        lanes = [lax.iota(jnp.int32, L) + g * L for g in range(ng)]

        def process(s, k, first):
            wait_gather(s, k)
            if not first:
                pltpu.make_async_copy(
                    obufs[k], out_slice(s - 2), osems[k]).wait()

            @plsc.parallel_loop(0, _D, unroll=8)
            def transpose(c):
                cvec = jnp.zeros((L,), jnp.int32) + c
                for g in range(ng):
                    val = plsc.load_gather(bufs[k], [lanes[g], cvec])
                    obufs[k][c, pl.ds(g * L, L)] = val

            pltpu.async_copy(obufs[k], out_slice(s), osems[k])

        fire_gather(0, 0)

        @pl.loop(0, 2, step=2)
        def head(s):
            fire_gather(s + 1, 1)
            process(s, 0, True)
            fire_gather(s + 2, 0)
            process(s + 1, 1, True)

        @pl.loop(2, seq - 2, step=2)
        def step(s):
            fire_gather(s + 1, 1)
            process(s, 0, False)
            fire_gather(s + 2, 0)
            process(s + 1, 1, False)

        @pl.loop(seq - 2, seq, step=2)
        def tail(s):
            fire_gather(s + 1, 1)
            process(s, 0, False)
            process(s + 1, 1, False)

        for s in range(seq - 2, seq):
            k = s % 2
            pltpu.make_async_copy(obufs[k], out_slice(s), osems[k]).wait()

    return emb


def kernel(batch_inputs, weight):
    bsz, seq = batch_inputs.shape
    V = weight.shape[0]
    wide = _widen_scale(weight.T)
    idxt = batch_inputs.astype(jnp.int32).T
    out3 = _make_gather(bsz, seq, V)(idxt, wide)
    return out3.transpose(2, 0, 1)


# speedup vs baseline: 1.0376x; 1.0376x over previous
import functools
import jax
import jax.numpy as jnp
from jax import lax
from jax.experimental import pallas as pl
from jax.experimental.pallas import tpu as pltpu
from jax.experimental.pallas import tpu_sc as plsc

_NC = 2
_NS = 16
_NW = _NC * _NS
_D = 64
_SCALE = 8.0
_IDXROW = 128
_NBUF = 4


def _make_kernel(B):
    bpw = B // _NW
    nchunk = bpw // _IDXROW

    mesh = plsc.VectorSubcoreMesh(
        core_axis_name="c", subcore_axis_name="s",
        num_cores=_NC, num_subcores=_NS)

    @functools.partial(
        pl.kernel,
        out_type=jax.ShapeDtypeStruct((B, _D), jnp.float32),
        mesh=mesh,
        scratch_types=[
            pltpu.VMEM((nchunk, _IDXROW), jnp.int32),
            [pltpu.VMEM((_IDXROW, _D), jnp.float32)] * _NBUF,
            [pltpu.SemaphoreType.DMA] * _NBUF,
            [pltpu.SemaphoreType.DMA] * _NBUF,
        ],
        compiler_params=pltpu.CompilerParams(use_tc_tiling_on_sc=False),
    )
    def emb(idx_hbm, table_hbm, out_hbm, idx_v, bufs, gsems, osems):
        wid = lax.axis_index("s") * _NC + lax.axis_index("c")
        base = wid * bpw
        pltpu.sync_copy(idx_hbm.at[wid], idx_v)

        def fire_gather(j, b):
            pltpu.async_copy(table_hbm.at[idx_v.at[j]], bufs[b], gsems[b])

        def wait_gather(j, b):
            pltpu.make_async_copy(
                table_hbm.at[idx_v.at[j]], bufs[b], gsems[b]).wait()

        def out_slice(j):
            return out_hbm.at[pl.ds(base + j * _IDXROW, _IDXROW)]

        for b in range(_NBUF - 1):
            fire_gather(b, b)

        @pl.loop(0, nchunk, step=_NBUF)
        def step(c):
            for db in range(_NBUF):
                j = c + db
                slot = db
                pb = (db + _NBUF - 1) % _NBUF
                wait_gather(j, slot)

                @pl.loop(0, _IDXROW)
                def scale_loop(r):
                    for u in range(_D // 16):
                        s = pl.ds(u * 16, 16)
                        bufs[slot][r, s] = bufs[slot][r, s] * _SCALE

                pltpu.async_copy(bufs[slot], out_slice(j), osems[slot])

                @pl.when(j + _NBUF - 1 < nchunk)
                def _():
                    @pl.when(j >= 1)
                    def _():
                        pltpu.make_async_copy(
                            bufs[pb], out_slice(j - 1), osems[pb]).wait()
                    fire_gather(j + _NBUF - 1, pb)

        for j in range(nchunk - _NBUF, nchunk):
            slot = j % _NBUF
            pltpu.make_async_copy(bufs[slot], out_slice(j), osems[slot]).wait()

    return emb


def kernel(batch_inputs, weight):
    bsz, seq = batch_inputs.shape
    B = bsz * seq
    idx = batch_inputs.astype(jnp.int32).reshape(_NW, B // (_NW * _IDXROW), _IDXROW)
    out = _make_kernel(B)(idx, weight)
    return out.reshape(bsz, seq, _D)
